# baseline (speedup 1.0000x reference)
import jax
import jax.numpy as jnp
from jax import lax
from jax.experimental import pallas as pl
from jax.experimental.pallas import tpu as pltpu

NEG_SLOPE = 0.2
DILATIONS = (1, 2, 4, 8)
G = 8
PRB = 8
VMEM_LIMIT = 32 * 1024 * 1024


def _leaky(x):
    return jnp.where(x >= 0, x, NEG_SLOPE * x)


def _to_slab(img, H, W):
    C = img.shape[0]
    gap = jnp.zeros((C, H, G), jnp.float32)
    return jnp.concatenate([img, gap], axis=2).reshape(C, H * (W + G))


def _mask(M, Wd, W):
    col = lax.broadcasted_iota(jnp.int32, (1, M), 1) % Wd
    return (col < W).astype(jnp.float32)


def _block(x_slab, wb, wm, C, H, W, maskf):
    Wd = W + G
    M = H * Wd
    L = PRB * Wd + G
    T = (H + 2 * PRB) * Wd + 2 * G

    xpf = jnp.concatenate(
        [jnp.zeros((C, L), jnp.bfloat16), x_slab.astype(jnp.bfloat16),
         jnp.zeros((C, T - L - M), jnp.bfloat16)], axis=1)

    slabs = []
    for d in DILATIONS:
        for ky in range(3):
            for kx in range(3):
                s = L + (ky - 1) * d * Wd + (kx - 1) * d
                slabs.append(xpf[:, s:s + M])
    patch = jnp.concatenate(slabs, axis=0)
    yb = jnp.dot(wb, patch, preferred_element_type=jnp.float32)
    yb = (_leaky(yb) * maskf).astype(jnp.bfloat16)

    Ly = Wd + 1
    Ty = M + 2 * Wd + 2
    ypf = jnp.concatenate(
        [jnp.zeros((2 * C, Ly), jnp.bfloat16), yb,
         jnp.zeros((2 * C, Ty - Ly - M), jnp.bfloat16)], axis=1)
    mslabs = []
    for ky in range(3):
        for kx in range(3):
            s = Ly + (ky - 1) * Wd + (kx - 1)
            mslabs.append(ypf[:, s:s + M])
    mpatch = jnp.concatenate(mslabs, axis=0)
    out = jnp.dot(wm, mpatch, preferred_element_type=jnp.float32)
    return _leaky(out + x_slab) * maskf


def _upsample(slab, wup, ucol, h, w):
    Cout = wup.shape[0]
    Wd = w + G
    Wo = 2 * w + G
    z = jnp.dot(wup, slab, preferred_element_type=jnp.float32)
    z = z.reshape(Cout, h, Wd)
    prev_r = jnp.concatenate([z[:, :1], z[:, :-1]], axis=1)
    next_r = jnp.concatenate([z[:, 1:], z[:, -1:]], axis=1)
    even_r = 0.25 * prev_r + 0.75 * z
    odd_r = 0.75 * z + 0.25 * next_r
    zr = jnp.stack([even_r, odd_r], axis=2).reshape(Cout, 2 * h, Wd)
    zf = zr.reshape(Cout * 2 * h, Wd)
    up = jnp.dot(zf, ucol, preferred_element_type=jnp.float32)
    return up.reshape(Cout, 2 * h, Wo)


def _decoder_kernel(e1_ref, e2_ref, e3_ref,
                    d1b0_wb, d1b0_wm, d1b1_wb, d1b1_wm,
                    d2b0_wb, d2b0_wm, d2b1_wb, d2b1_wm,
                    d3b0_wb, d3b0_wm, d3b1_wb, d3b1_wm,
                    sa1_wb, sa1_wm, sa2_wb, sa2_wm,
                    up21_w, up32_w, u32_col, u21_col, o_ref):
    B, C1, H1, W1 = e1_ref.shape
    _, C2, H2, W2 = e2_ref.shape
    _, C3, H3, W3 = e3_ref.shape
    m1 = _mask(H1 * (W1 + G), W1 + G, W1)
    m2 = _mask(H2 * (W2 + G), W2 + G, W2)
    m3 = _mask(H3 * (W3 + G), W3 + G, W3)

    for i in range(B):
        d3 = _to_slab(e3_ref[i].astype(jnp.float32), H3, W3)
        d3 = _block(d3, d3b0_wb[...], d3b0_wm[...], C3, H3, W3, m3)
        d3 = _block(d3, d3b1_wb[...], d3b1_wm[...], C3, H3, W3, m3)

        s2 = _to_slab(e2_ref[i].astype(jnp.float32), H2, W2)
        s2 = _block(s2, sa2_wb[...], sa2_wm[...], C2, H2, W2, m2)
        d2 = (_upsample(d3, up32_w[...], u32_col[...], H3, W3)
              + s2.reshape(C2, H2, W2 + G)).reshape(C2, H2 * (W2 + G))
        d2 = _block(d2, d2b0_wb[...], d2b0_wm[...], C2, H2, W2, m2)
        d2 = _block(d2, d2b1_wb[...], d2b1_wm[...], C2, H2, W2, m2)

        s1 = _to_slab(e1_ref[i].astype(jnp.float32), H1, W1)
        s1 = _block(s1, sa1_wb[...], sa1_wm[...], C1, H1, W1, m1)
        d1 = (_upsample(d2, up21_w[...], u21_col[...], H2, W2)
              + s1.reshape(C1, H1, W1 + G)).reshape(C1, H1 * (W1 + G))
        d1 = _block(d1, d1b0_wb[...], d1b0_wm[...], C1, H1, W1, m1)
        d1 = _block(d1, d1b1_wb[...], d1b1_wm[...], C1, H1, W1, m1)

        o_ref[i] = d1.reshape(C1, H1, W1 + G)[:, :, :W1].astype(o_ref.dtype)


def _make_ucol(w):
    Wd, Wo = w + G, 2 * w + G
    k = jnp.arange(Wd)[:, None]
    c = jnp.arange(Wo)[None, :]
    j = c // 2
    odd = c % 2
    jsel = jnp.where(odd == 1, jnp.minimum(j + 1, w - 1), jnp.maximum(j - 1, 0))
    u = 0.75 * (k == j) + 0.25 * (k == jsel)
    return jnp.where(c < 2 * w, u, 0.0).astype(jnp.float32)


def kernel(enc1, enc2, enc3, d1b0_wb, d1b0_wm, d1b1_wb, d1b1_wm,
           d2b0_wb, d2b0_wm, d2b1_wb, d2b1_wm, d3b0_wb, d3b0_wm,
           d3b1_wb, d3b1_wm, sa1_wb, sa1_wm, sa2_wb, sa2_wm,
           up21_w, up32_w):
    N, C1, H1, W1 = enc1.shape
    _, C2, H2, W2 = enc2.shape
    _, C3, H3, W3 = enc3.shape

    bf = lambda a: a.astype(jnp.bfloat16)
    ws = [bf(d1b0_wb), bf(d1b0_wm), bf(d1b1_wb), bf(d1b1_wm),
          bf(d2b0_wb), bf(d2b0_wm), bf(d2b1_wb), bf(d2b1_wm),
          bf(d3b0_wb), bf(d3b0_wm), bf(d3b1_wb), bf(d3b1_wm),
          bf(sa1_wb), bf(sa1_wm), bf(sa2_wb), bf(sa2_wm),
          up21_w, up32_w, _make_ucol(W3), _make_ucol(W2)]

    B = 2 if N % 2 == 0 else 1
    img = lambda c, h, w: pl.BlockSpec((B, c, h, w),
                                       lambda n: (n, 0, 0, 0))
    full = lambda a: pl.BlockSpec(a.shape, lambda n: (0,) * a.ndim)

    flops = 0
    for c, h, w, nblk in ((C1, H1, W1, 3), (C2, H2, W2, 3), (C3, H3, W3, 2)):
        flops += nblk * 2 * N * h * (w + G) * (90 * c * c)
    bytes_accessed = 4 * (enc1.size + enc2.size + enc3.size + enc1.size)

    return pl.pallas_call(
        _decoder_kernel,
        out_shape=jax.ShapeDtypeStruct((N, C1, H1, W1), enc1.dtype),
        grid=(N // B,),
        in_specs=[img(C1, H1, W1), img(C2, H2, W2), img(C3, H3, W3)]
        + [full(a) for a in ws],
        out_specs=img(C1, H1, W1),
        compiler_params=pltpu.CompilerParams(
            dimension_semantics=("parallel",),
            vmem_limit_bytes=VMEM_LIMIT),
        cost_estimate=pl.CostEstimate(flops=int(flops), transcendentals=0,
                                      bytes_accessed=int(bytes_accessed)),
    )(enc1, enc2, enc3, *ws)


# speedup vs baseline: 1.2045x; 1.2045x over previous
import jax
import jax.numpy as jnp
from jax import lax
from jax.experimental import pallas as pl
from jax.experimental.pallas import tpu as pltpu

NEG_SLOPE = 0.2
DILATIONS = (1, 2, 4, 8)
G = 8
PRB = 8
VMEM_LIMIT = 32 * 1024 * 1024


def _leaky(x):
    return jnp.where(x >= 0, x, NEG_SLOPE * x)


def _to_slab(img, H, W):
    C = img.shape[0]
    gap = jnp.zeros((C, H, G), jnp.float32)
    return jnp.concatenate([img, gap], axis=2).reshape(C, H * (W + G))


def _mask(M, Wd, W):
    col = lax.broadcasted_iota(jnp.int32, (1, M), 1) % Wd
    return (col < W).astype(jnp.float32)


def _block(x_slab, wb, wm, C, H, W, maskf):
    Wd = W + G
    M = H * Wd
    L = PRB * Wd + G
    T = (H + 2 * PRB) * Wd + 2 * G

    xpf = jnp.concatenate(
        [jnp.zeros((C, L), jnp.bfloat16), x_slab.astype(jnp.bfloat16),
         jnp.zeros((C, T - L - M), jnp.bfloat16)], axis=1)

    slabs = []
    for d in DILATIONS:
        for ky in range(3):
            for kx in range(3):
                s = L + (ky - 1) * d * Wd + (kx - 1) * d
                slabs.append(xpf[:, s:s + M])
    patch = jnp.concatenate(slabs, axis=0)
    yb = jnp.dot(wb, patch, preferred_element_type=jnp.float32)
    yb = (_leaky(yb) * maskf).astype(jnp.bfloat16)

    Ly = Wd + 1
    Ty = M + 2 * Wd + 2
    ypf = jnp.concatenate(
        [jnp.zeros((2 * C, Ly), jnp.bfloat16), yb,
         jnp.zeros((2 * C, Ty - Ly - M), jnp.bfloat16)], axis=1)
    mslabs = []
    for ky in range(3):
        for kx in range(3):
            s = Ly + (ky - 1) * Wd + (kx - 1)
            mslabs.append(ypf[:, s:s + M])
    mpatch = jnp.concatenate(mslabs, axis=0)
    out = jnp.dot(wm, mpatch, preferred_element_type=jnp.float32)
    return _leaky(out + x_slab) * maskf


def _upsample(slab, wup, ucol, h, w):
    Cout = wup.shape[0]
    Wd = w + G
    Wo = 2 * w + G
    z = jnp.dot(wup, slab, preferred_element_type=jnp.float32)
    z = z.reshape(Cout, h, Wd)
    prev_r = jnp.concatenate([z[:, :1], z[:, :-1]], axis=1)
    next_r = jnp.concatenate([z[:, 1:], z[:, -1:]], axis=1)
    even_r = 0.25 * prev_r + 0.75 * z
    odd_r = 0.75 * z + 0.25 * next_r
    zr = jnp.stack([even_r, odd_r], axis=2).reshape(Cout, 2 * h, Wd)
    zf = zr.reshape(Cout * 2 * h, Wd)
    up = jnp.dot(zf, ucol, preferred_element_type=jnp.float32)
    return up.reshape(Cout, 2 * h, Wo)


def _decoder_kernel(e1_ref, e2_ref, e3_ref,
                    d1b0_wb, d1b0_wm, d1b1_wb, d1b1_wm,
                    d2b0_wb, d2b0_wm, d2b1_wb, d2b1_wm,
                    d3b0_wb, d3b0_wm, d3b1_wb, d3b1_wm,
                    sa1_wb, sa1_wm, sa2_wb, sa2_wm,
                    up21_w, up32_w, u32_col, u21_col, o_ref):
    B, C1, H1, W1 = e1_ref.shape
    _, C2, H2, W2 = e2_ref.shape
    _, C3, H3, W3 = e3_ref.shape
    m1 = _mask(H1 * (W1 + G), W1 + G, W1)
    m2 = _mask(H2 * (W2 + G), W2 + G, W2)
    m3 = _mask(H3 * (W3 + G), W3 + G, W3)

    for i in range(B):
        d3 = _to_slab(e3_ref[i].astype(jnp.float32), H3, W3)
        d3 = _block(d3, d3b0_wb[...], d3b0_wm[...], C3, H3, W3, m3)
        d3 = _block(d3, d3b1_wb[...], d3b1_wm[...], C3, H3, W3, m3)

        s2 = _to_slab(e2_ref[i].astype(jnp.float32), H2, W2)
        s2 = _block(s2, sa2_wb[...], sa2_wm[...], C2, H2, W2, m2)
        d2 = (_upsample(d3, up32_w[...], u32_col[...], H3, W3)
              + s2.reshape(C2, H2, W2 + G)).reshape(C2, H2 * (W2 + G))
        d2 = _block(d2, d2b0_wb[...], d2b0_wm[...], C2, H2, W2, m2)
        d2 = _block(d2, d2b1_wb[...], d2b1_wm[...], C2, H2, W2, m2)

        s1 = _to_slab(e1_ref[i].astype(jnp.float32), H1, W1)
        s1 = _block(s1, sa1_wb[...], sa1_wm[...], C1, H1, W1, m1)
        d1 = (_upsample(d2, up21_w[...], u21_col[...], H2, W2)
              + s1.reshape(C1, H1, W1 + G)).reshape(C1, H1 * (W1 + G))
        d1 = _block(d1, d1b0_wb[...], d1b0_wm[...], C1, H1, W1, m1)
        d1 = _block(d1, d1b1_wb[...], d1b1_wm[...], C1, H1, W1, m1)

        o_ref[i] = d1.reshape(C1, H1, W1 + G)[:, :, :W1].astype(o_ref.dtype)


def _make_ucol(w):
    Wd, Wo = w + G, 2 * w + G
    k = jnp.arange(Wd)[:, None]
    c = jnp.arange(Wo)[None, :]
    j = c // 2
    odd = c % 2
    jsel = jnp.where(odd == 1, jnp.minimum(j + 1, w - 1), jnp.maximum(j - 1, 0))
    u = 0.75 * (k == j) + 0.25 * (k == jsel)
    return jnp.where(c < 2 * w, u, 0.0).astype(jnp.float32)


def kernel(enc1, enc2, enc3, d1b0_wb, d1b0_wm, d1b1_wb, d1b1_wm,
           d2b0_wb, d2b0_wm, d2b1_wb, d2b1_wm, d3b0_wb, d3b0_wm,
           d3b1_wb, d3b1_wm, sa1_wb, sa1_wm, sa2_wb, sa2_wm,
           up21_w, up32_w):
    N, C1, H1, W1 = enc1.shape
    _, C2, H2, W2 = enc2.shape
    _, C3, H3, W3 = enc3.shape

    bf = lambda a: a.astype(jnp.bfloat16)
    ws = [bf(d1b0_wb), bf(d1b0_wm), bf(d1b1_wb), bf(d1b1_wm),
          bf(d2b0_wb), bf(d2b0_wm), bf(d2b1_wb), bf(d2b1_wm),
          bf(d3b0_wb), bf(d3b0_wm), bf(d3b1_wb), bf(d3b1_wm),
          bf(sa1_wb), bf(sa1_wm), bf(sa2_wb), bf(sa2_wm),
          up21_w, up32_w, _make_ucol(W3), _make_ucol(W2)]

    B = 1
    img = lambda c, h, w: pl.BlockSpec((B, c, h, w),
                                       lambda n: (n, 0, 0, 0))
    full = lambda a: pl.BlockSpec(a.shape, lambda n: (0,) * a.ndim)

    flops = 0
    for c, h, w, nblk in ((C1, H1, W1, 3), (C2, H2, W2, 3), (C3, H3, W3, 2)):
        flops += nblk * 2 * N * h * (w + G) * (90 * c * c)
    bytes_accessed = 4 * (enc1.size + enc2.size + enc3.size + enc1.size)

    return pl.pallas_call(
        _decoder_kernel,
        out_shape=jax.ShapeDtypeStruct((N, C1, H1, W1), enc1.dtype),
        grid=(N // B,),
        in_specs=[img(C1, H1, W1), img(C2, H2, W2), img(C3, H3, W3)]
        + [full(a) for a in ws],
        out_specs=img(C1, H1, W1),
        compiler_params=pltpu.CompilerParams(
            dimension_semantics=("parallel",),
            vmem_limit_bytes=VMEM_LIMIT),
        cost_estimate=pl.CostEstimate(flops=int(flops), transcendentals=0,
                                      bytes_accessed=int(bytes_accessed)),
    )(enc1, enc2, enc3, *ws)
